# baseline (speedup 1.0000x reference)
import functools

import jax
import jax.numpy as jnp
from jax import lax
from jax.experimental import pallas as pl
from jax.experimental.pallas import tpu as pltpu
from jax.experimental.pallas import tpu_sc as plsc

N_FIELDS = 26
EMBED_DIM = 16
BATCH = 4096
FIELD_DIM = 4000
TOTAL = FIELD_DIM * N_FIELDS

NC, NS, L = 2, 16, 16
TAB_PER_SC = N_FIELDS // NC


def _sc_body(
    wt_hbm, xt_hbm, out_hbm, vrow_v, xrow_v, orow_v, vsem, xsems, osems
):
    c = lax.axis_index("c")
    t = lax.axis_index("s")

    @pl.loop(0, TAB_PER_SC)
    def _tab(j):
        i = j * NC + c
        vcp = pltpu.async_copy(wt_hbm.at[i, t, :], vrow_v, vsem)
        for p in range(2):
            pltpu.async_copy(xt_hbm.at[p, :], xrow_v.at[p], xsems[p])

        @pl.when(j > 0)
        def _drain_tail():
            for p in range(2):
                pltpu.make_async_copy(
                    orow_v.at[p], out_hbm.at[i, p, t, :], osems[p]
                ).wait()

        vcp.wait()

        @pl.loop(0, N_FIELDS, step=2)
        def _field2(f0):
            for p in range(2):
                f = f0 + p
                pltpu.make_async_copy(
                    xt_hbm.at[f, :], xrow_v.at[p], xsems[p]
                ).wait()

                @pl.when(f0 > 0)
                def _drain_prev():
                    pltpu.make_async_copy(
                        orow_v.at[p], out_hbm.at[i, f, t, :], osems[p]
                    ).wait()

                offv = jnp.full((L,), f * FIELD_DIM, jnp.int32)

                @plsc.parallel_loop(0, BATCH, step=L, unroll=8)
                def _gather(k):
                    s = pl.ds(k, L)
                    iv = xrow_v[p, s] + offv
                    orow_v[p, s] = plsc.load_gather(vrow_v, [iv])

                @pl.when(f + 2 < N_FIELDS)
                def _next_idx():
                    pltpu.async_copy(xt_hbm.at[f + 2, :], xrow_v.at[p], xsems[p])

                pltpu.async_copy(orow_v.at[p], out_hbm.at[i, f, t, :], osems[p])

    for p in range(2):
        pltpu.make_async_copy(
            orow_v.at[p], out_hbm.at[0, p, t, :], osems[p]
        ).wait()


@functools.lru_cache(maxsize=1)
def _build_call():
    mesh = plsc.VectorSubcoreMesh(
        core_axis_name="c", subcore_axis_name="s", num_cores=NC, num_subcores=NS
    )
    return pl.kernel(
        _sc_body,
        out_type=jax.ShapeDtypeStruct(
            (N_FIELDS, N_FIELDS, EMBED_DIM, BATCH), jnp.float32
        ),
        mesh=mesh,
        scratch_types=[
            pltpu.VMEM((TOTAL,), jnp.float32),
            pltpu.VMEM((2, BATCH), jnp.int32),
            pltpu.VMEM((2, BATCH), jnp.float32),
            pltpu.SemaphoreType.DMA,
            [pltpu.SemaphoreType.DMA] * 2,
            [pltpu.SemaphoreType.DMA] * 2,
        ],
        compiler_params=pltpu.CompilerParams(
            use_tc_tiling_on_sc=True, needs_layout_passes=False
        ),
    )


def kernel(x, W):
    wt = jnp.transpose(W, (0, 2, 1))
    xt = jnp.transpose(x, (1, 0))
    out = _build_call()(wt, xt)
    return jnp.transpose(out, (0, 3, 1, 2))


# speedup vs baseline: 1.1655x; 1.1655x over previous
import functools

import jax
import jax.numpy as jnp
from jax import lax
from jax.experimental import pallas as pl
from jax.experimental.pallas import tpu as pltpu
from jax.experimental.pallas import tpu_sc as plsc

N_FIELDS = 26
EMBED_DIM = 16
BATCH = 4096
FIELD_DIM = 4000
TOTAL = FIELD_DIM * N_FIELDS

NC, NS, L = 2, 16, 16
TAB_PER_SC = N_FIELDS // NC


def _sc_body(
    wt_hbm, xt_hbm, out_hbm, vrow_v, xrows_v, orow_v, vsem, xsems, osems
):
    c = lax.axis_index("c")
    t = lax.axis_index("s")

    @pl.loop(0, TAB_PER_SC)
    def _tab(j):
        i = j * NC + c
        vcp = pltpu.async_copy(wt_hbm.at[i, t, :], vrow_v, vsem)
        for p in range(2):
            pltpu.async_copy(xt_hbm.at[p, :], xrows_v[p], xsems[p])

        @pl.when(j > 0)
        def _drain_tail():
            for p in range(2):
                pltpu.make_async_copy(
                    orow_v.at[p], out_hbm.at[i, p, t, :], osems[p]
                ).wait()

        vcp.wait()

        @pl.loop(0, N_FIELDS, step=2)
        def _field2(f0):
            for p in range(2):
                f = f0 + p
                pltpu.make_async_copy(
                    xt_hbm.at[f, :], xrows_v[p], xsems[p]
                ).wait()

                @pl.when(f0 > 0)
                def _drain_prev():
                    pltpu.make_async_copy(
                        orow_v.at[p], out_hbm.at[i, f, t, :], osems[p]
                    ).wait()

                offv = jnp.full((L,), f * FIELD_DIM, jnp.int32)

                @plsc.parallel_loop(0, BATCH // (2 * L), unroll=8)
                def _gather(g):
                    vi = xrows_v[p][pl.ds(g * L, L)]
                    lo = (vi & jnp.int32(0xFFFF)) + offv
                    hi = lax.shift_right_logical(vi, jnp.int32(16)) + offv
                    orow_v[p, pl.ds(g * 2 * L, L)] = plsc.load_gather(vrow_v, [lo])
                    orow_v[p, pl.ds(g * 2 * L + L, L)] = plsc.load_gather(vrow_v, [hi])

                @pl.when(f + 2 < N_FIELDS)
                def _next_idx():
                    pltpu.async_copy(xt_hbm.at[f + 2, :], xrows_v[p], xsems[p])

                pltpu.async_copy(orow_v.at[p], out_hbm.at[i, f, t, :], osems[p])

    for p in range(2):
        pltpu.make_async_copy(
            orow_v.at[p], out_hbm.at[0, p, t, :], osems[p]
        ).wait()


@functools.lru_cache(maxsize=1)
def _build_call():
    mesh = plsc.VectorSubcoreMesh(
        core_axis_name="c", subcore_axis_name="s", num_cores=NC, num_subcores=NS
    )
    return pl.kernel(
        _sc_body,
        out_type=jax.ShapeDtypeStruct(
            (N_FIELDS, N_FIELDS, EMBED_DIM, BATCH), jnp.float32
        ),
        mesh=mesh,
        scratch_types=[
            pltpu.VMEM((TOTAL,), jnp.float32),
            [pltpu.VMEM((BATCH // 2,), jnp.int32)] * 2,
            pltpu.VMEM((2, BATCH), jnp.float32),
            pltpu.SemaphoreType.DMA,
            [pltpu.SemaphoreType.DMA] * 2,
            [pltpu.SemaphoreType.DMA] * 2,
        ],
        compiler_params=pltpu.CompilerParams(
            use_tc_tiling_on_sc=True, needs_layout_passes=False
        ),
    )


def kernel(x, W):
    wt = jnp.transpose(W, (0, 2, 1))
    xi = jnp.transpose(x, (1, 0)).reshape(N_FIELDS, BATCH // (2 * L), 2, L)
    xs = (xi[:, :, 0, :] | (xi[:, :, 1, :] << 16)).reshape(N_FIELDS, BATCH // 2)
    out = _build_call()(wt, xs)
    return jnp.transpose(out, (0, 3, 1, 2))


# speedup vs baseline: 1.1695x; 1.0035x over previous
import functools

import jax
import jax.numpy as jnp
from jax import lax
from jax.experimental import pallas as pl
from jax.experimental.pallas import tpu as pltpu
from jax.experimental.pallas import tpu_sc as plsc

N_FIELDS = 26
EMBED_DIM = 16
BATCH = 4096
FIELD_DIM = 4000
TOTAL = FIELD_DIM * N_FIELDS

NC, NS, L = 2, 16, 16
TAB_PER_SC = N_FIELDS // NC


def _sc_body(
    wt_hbm, xt_hbm, out_hbm, vrow_v, xrows_v, orow_v, vsem, xsems, osems
):
    c = lax.axis_index("c")
    t = lax.axis_index("s")

    @pl.loop(0, TAB_PER_SC)
    def _tab(j):
        i = j * NC + c
        vcp = pltpu.async_copy(wt_hbm.at[i, t, :], vrow_v, vsem)
        for p in range(2):
            pltpu.async_copy(xt_hbm.at[p, :], xrows_v[p], xsems[p])

        @pl.when(j > 0)
        def _drain_tail():
            for p in range(2):
                pltpu.make_async_copy(
                    orow_v.at[p], out_hbm.at[i, p, t, :], osems[p]
                ).wait()

        vcp.wait()

        @pl.loop(0, N_FIELDS, step=2)
        def _field2(f0):
            for p in range(2):
                f = f0 + p
                pltpu.make_async_copy(
                    xt_hbm.at[f, :], xrows_v[p], xsems[p]
                ).wait()

                @pl.when(f0 > 0)
                def _drain_prev():
                    pltpu.make_async_copy(
                        orow_v.at[p], out_hbm.at[i, f, t, :], osems[p]
                    ).wait()

                offv = jnp.full((L,), f * FIELD_DIM, jnp.int32)

                @plsc.parallel_loop(0, BATCH // (2 * L), unroll=16)
                def _gather(g):
                    vi = xrows_v[p][pl.ds(g * L, L)]
                    lo = (vi & jnp.int32(0xFFFF)) + offv
                    hi = lax.shift_right_logical(vi, jnp.int32(16)) + offv
                    orow_v[p, pl.ds(g * 2 * L, L)] = plsc.load_gather(vrow_v, [lo])
                    orow_v[p, pl.ds(g * 2 * L + L, L)] = plsc.load_gather(vrow_v, [hi])

                @pl.when(f + 2 < N_FIELDS)
                def _next_idx():
                    pltpu.async_copy(xt_hbm.at[f + 2, :], xrows_v[p], xsems[p])

                pltpu.async_copy(orow_v.at[p], out_hbm.at[i, f, t, :], osems[p])

    for p in range(2):
        pltpu.make_async_copy(
            orow_v.at[p], out_hbm.at[0, p, t, :], osems[p]
        ).wait()


@functools.lru_cache(maxsize=1)
def _build_call():
    mesh = plsc.VectorSubcoreMesh(
        core_axis_name="c", subcore_axis_name="s", num_cores=NC, num_subcores=NS
    )
    return pl.kernel(
        _sc_body,
        out_type=jax.ShapeDtypeStruct(
            (N_FIELDS, N_FIELDS, EMBED_DIM, BATCH), jnp.float32
        ),
        mesh=mesh,
        scratch_types=[
            pltpu.VMEM((TOTAL,), jnp.float32),
            [pltpu.VMEM((BATCH // 2,), jnp.int32)] * 2,
            pltpu.VMEM((2, BATCH), jnp.float32),
            pltpu.SemaphoreType.DMA,
            [pltpu.SemaphoreType.DMA] * 2,
            [pltpu.SemaphoreType.DMA] * 2,
        ],
        compiler_params=pltpu.CompilerParams(
            use_tc_tiling_on_sc=True, needs_layout_passes=False
        ),
    )


def kernel(x, W):
    wt = jnp.transpose(W, (0, 2, 1))
    xi = jnp.transpose(x, (1, 0)).reshape(N_FIELDS, BATCH // (2 * L), 2, L)
    xs = (xi[:, :, 0, :] | (xi[:, :, 1, :] << 16)).reshape(N_FIELDS, BATCH // 2)
    out = _build_call()(wt, xs)
    return jnp.transpose(out, (0, 3, 1, 2))


# speedup vs baseline: 1.2863x; 1.0999x over previous
import functools

import jax
import jax.numpy as jnp
from jax import lax
from jax.experimental import pallas as pl
from jax.experimental.pallas import tpu as pltpu
from jax.experimental.pallas import tpu_sc as plsc

N_FIELDS = 26
EMBED_DIM = 16
BATCH = 4096
FIELD_DIM = 4000
TOTAL = FIELD_DIM * N_FIELDS

NC, NS, L = 2, 16, 16
TAB_PER_SC = N_FIELDS // NC


def _sc_body(
    wt_hbm, xt_hbm, out_hbm, vrow_v, xrows_v, orow_v, sp_x, vsem, xsems, osems
):
    c = lax.axis_index("c")
    t = lax.axis_index("s")

    pltpu.sync_copy(xt_hbm.at[t], sp_x.at[t])

    @pl.when(t < N_FIELDS - NS)
    def _stage_rest():
        pltpu.sync_copy(xt_hbm.at[t + NS], sp_x.at[t + NS])

    plsc.subcore_barrier()

    @pl.loop(0, TAB_PER_SC)
    def _tab(j):
        i = j * NC + c
        vcp = pltpu.async_copy(wt_hbm.at[i, t, :], vrow_v, vsem)
        for p in range(2):
            pltpu.async_copy(sp_x.at[p], xrows_v[p], xsems[p])

        @pl.when(j > 0)
        def _drain_tail():
            for p in range(2):
                pltpu.make_async_copy(
                    orow_v.at[p], out_hbm.at[i, p, t, :], osems[p]
                ).wait()

        vcp.wait()

        @pl.loop(0, N_FIELDS, step=2)
        def _field2(f0):
            for p in range(2):
                f = f0 + p
                pltpu.make_async_copy(
                    sp_x.at[f], xrows_v[p], xsems[p]
                ).wait()

                @pl.when(f0 > 0)
                def _drain_prev():
                    pltpu.make_async_copy(
                        orow_v.at[p], out_hbm.at[i, f, t, :], osems[p]
                    ).wait()

                offv = jnp.full((L,), f * FIELD_DIM, jnp.int32)

                @plsc.parallel_loop(0, L, unroll=2)
                def _gatherq(q):
                    for r in range(8):
                        vi = xrows_v[p][q, pl.ds(r * L, L)]
                        lo = (vi & jnp.int32(0xFFFF)) + offv
                        hi = lax.shift_right_logical(vi, jnp.int32(16)) + offv
                        b0 = q * 16 * L + r * 2 * L
                        orow_v[p, pl.ds(b0, L)] = plsc.load_gather(vrow_v, [lo])
                        orow_v[p, pl.ds(b0 + L, L)] = plsc.load_gather(vrow_v, [hi])

                @pl.when(f + 2 < N_FIELDS)
                def _next_idx():
                    pltpu.async_copy(sp_x.at[f + 2], xrows_v[p], xsems[p])

                pltpu.async_copy(orow_v.at[p], out_hbm.at[i, f, t, :], osems[p])

    for p in range(2):
        pltpu.make_async_copy(
            orow_v.at[p], out_hbm.at[0, p, t, :], osems[p]
        ).wait()


@functools.lru_cache(maxsize=1)
def _build_call():
    mesh = plsc.VectorSubcoreMesh(
        core_axis_name="c", subcore_axis_name="s", num_cores=NC, num_subcores=NS
    )
    return pl.kernel(
        _sc_body,
        out_type=jax.ShapeDtypeStruct(
            (N_FIELDS, N_FIELDS, EMBED_DIM, BATCH), jnp.float32
        ),
        mesh=mesh,
        scratch_types=[
            pltpu.VMEM((TOTAL,), jnp.float32),
            [pltpu.VMEM((L, 2 * L * 4), jnp.int32)] * 2,
            pltpu.VMEM((2, BATCH), jnp.float32),
            pltpu.VMEM_SHARED((N_FIELDS, L, 2 * L * 4), jnp.int32),
            pltpu.SemaphoreType.DMA,
            [pltpu.SemaphoreType.DMA] * 2,
            [pltpu.SemaphoreType.DMA] * 2,
        ],
        compiler_params=pltpu.CompilerParams(
            use_tc_tiling_on_sc=True, needs_layout_passes=False
        ),
    )


def kernel(x, W):
    wt = jnp.transpose(W, (0, 2, 1))
    xi = jnp.transpose(x, (1, 0)).reshape(N_FIELDS, BATCH // (2 * L), 2, L)
    xs = (xi[:, :, 0, :] | (xi[:, :, 1, :] << 16)).reshape(N_FIELDS, L, 2 * L * 4)
    out = _build_call()(wt, xs)
    return jnp.transpose(out, (0, 3, 1, 2))


# speedup vs baseline: 1.3298x; 1.0338x over previous
import functools

import jax
import jax.numpy as jnp
from jax import lax
from jax.experimental import pallas as pl
from jax.experimental.pallas import tpu as pltpu
from jax.experimental.pallas import tpu_sc as plsc

N_FIELDS = 26
EMBED_DIM = 16
BATCH = 4096
FIELD_DIM = 4000
TOTAL = FIELD_DIM * N_FIELDS

NC, NS, L = 2, 16, 16
TAB_PER_SC = N_FIELDS // NC


def _sc_body(
    wt_hbm, xt_hbm, out_hbm, vrow_v, xrows_v, orow_v, sp_x, vsem, xsems, osems
):
    c = lax.axis_index("c")
    t = lax.axis_index("s")

    pltpu.sync_copy(xt_hbm.at[t], sp_x.at[t])

    @pl.when(t < N_FIELDS - NS)
    def _stage_rest():
        pltpu.sync_copy(xt_hbm.at[t + NS], sp_x.at[t + NS])

    plsc.subcore_barrier()

    @pl.loop(0, TAB_PER_SC)
    def _tab(j):
        i = j * NC + c
        vcp = pltpu.async_copy(wt_hbm.at[i, t, :], vrow_v, vsem)
        for p in range(2):
            pltpu.async_copy(sp_x.at[p], xrows_v[p], xsems[p])

        @pl.when(j > 0)
        def _drain_tail():
            for p in range(2):
                pltpu.make_async_copy(
                    orow_v.at[p], out_hbm.at[i, p, t, :], osems[p]
                ).wait()

        vcp.wait()

        @pl.loop(0, N_FIELDS, step=2)
        def _field2(f0):
            for p in range(2):
                f = f0 + p
                pltpu.make_async_copy(
                    sp_x.at[f], xrows_v[p], xsems[p]
                ).wait()

                @pl.when(f0 > 0)
                def _drain_prev():
                    pltpu.make_async_copy(
                        orow_v.at[p], out_hbm.at[i, f, t, :], osems[p]
                    ).wait()

                offv = jnp.full((L,), f * FIELD_DIM, jnp.int32)

                @plsc.parallel_loop(0, L, unroll=4)
                def _gatherq(q):
                    for r in range(8):
                        vi = xrows_v[p][q, pl.ds(r * L, L)]
                        lo = (vi & jnp.int32(0xFFFF)) + offv
                        hi = lax.shift_right_logical(vi, jnp.int32(16)) + offv
                        b0 = q * 16 * L + r * 2 * L
                        orow_v[p, pl.ds(b0, L)] = plsc.load_gather(vrow_v, [lo])
                        orow_v[p, pl.ds(b0 + L, L)] = plsc.load_gather(vrow_v, [hi])

                @pl.when(f + 2 < N_FIELDS)
                def _next_idx():
                    pltpu.async_copy(sp_x.at[f + 2], xrows_v[p], xsems[p])

                pltpu.async_copy(orow_v.at[p], out_hbm.at[i, f, t, :], osems[p])

    for p in range(2):
        pltpu.make_async_copy(
            orow_v.at[p], out_hbm.at[0, p, t, :], osems[p]
        ).wait()


@functools.lru_cache(maxsize=1)
def _build_call():
    mesh = plsc.VectorSubcoreMesh(
        core_axis_name="c", subcore_axis_name="s", num_cores=NC, num_subcores=NS
    )
    return pl.kernel(
        _sc_body,
        out_type=jax.ShapeDtypeStruct(
            (N_FIELDS, N_FIELDS, EMBED_DIM, BATCH), jnp.float32
        ),
        mesh=mesh,
        scratch_types=[
            pltpu.VMEM((TOTAL,), jnp.float32),
            [pltpu.VMEM((L, 2 * L * 4), jnp.int32)] * 2,
            pltpu.VMEM((2, BATCH), jnp.float32),
            pltpu.VMEM_SHARED((N_FIELDS, L, 2 * L * 4), jnp.int32),
            pltpu.SemaphoreType.DMA,
            [pltpu.SemaphoreType.DMA] * 2,
            [pltpu.SemaphoreType.DMA] * 2,
        ],
        compiler_params=pltpu.CompilerParams(
            use_tc_tiling_on_sc=True, needs_layout_passes=False
        ),
    )


def kernel(x, W):
    wt = jnp.transpose(W, (0, 2, 1))
    xi = jnp.transpose(x, (1, 0)).reshape(N_FIELDS, BATCH // (2 * L), 2, L)
    xs = (xi[:, :, 0, :] | (xi[:, :, 1, :] << 16)).reshape(N_FIELDS, L, 2 * L * 4)
    out = _build_call()(wt, xs)
    return jnp.transpose(out, (0, 3, 1, 2))
